# baseline (speedup 1.0000x reference)
import jax
import jax.numpy as jnp
from jax import lax
from jax.experimental import pallas as pl
from jax.experimental.pallas import tpu as pltpu
from jax.experimental.pallas import tpu_sc as plsc

B = 2
N = 10000
E = 320000
D = 128
H = 512
OUT = 5

NP = 10240
NR = NP // 128
NC = 2
NS = 16
EPT = E // NS
EBLK = 4000
NBLK = EPT // EBLK
GU = 5
NP2 = NP + 16
ZROWS = NP2 // NS
SLICE = NP // NS
WL = 2048


def _sc_mesh():
    return plsc.VectorSubcoreMesh(
        core_axis_name="c", subcore_axis_name="s", num_cores=NC, num_subcores=NS
    )


def _sc_params():
    return pltpu.CompilerParams(
        needs_layout_passes=False, use_tc_tiling_on_sc=False
    )


def _process16(c, src16, dst16, mk, xflat, sacc, dbuf, rowbuf, dsvb, fence,
               gxs, sem):
    gidx = jnp.where(mk, src16, 0) + c * NP
    pltpu.async_copy(xflat.at[gidx], rowbuf, sem).wait()
    dsv = plsc.load_gather(dbuf, [src16 >> 7, src16 & 127])
    plsc.addupdate_scatter(sacc, [dst16 >> 7, dst16 & 127], dsv, mask=mk)
    for r in range(16):
        sc16 = lax.gather(
            dsv,
            jnp.full((16, 1), r, jnp.int32),
            lax.GatherDimensionNumbers(
                offset_dims=(), collapsed_slice_dims=(0,),
                start_index_map=(0,)),
            (1,),
            mode=lax.GatherScatterMode.PROMISE_IN_BOUNDS,
        )
        for k in range(D // 16):
            rowbuf[r, pl.ds(k * 16, 16)] *= sc16
    pltpu.sync_copy(rowbuf, fence)
    didx = jnp.where(mk, dst16, NP)
    pltpu.sync_copy(rowbuf, gxs.at[didx], add=True)


def _mega_body(adj, xflat, zeros, gxr_out, dinv_out, ap_out, svec_out, fence,
               srcb, dstb, hA, hB, hC, rowbuf, dsvb, wl, idxb,
               degS, c0S, sS, gxs, sem):
    c = lax.axis_index("c")
    s = lax.axis_index("s")
    i16 = lax.broadcasted_iota(jnp.int32, (16,), 0)
    ones = jnp.full((16,), 1.0, jnp.float32)
    tmask = jnp.full((16,), True)

    pltpu.sync_copy(zeros.at[pl.ds(0, ZROWS), :], gxs.at[pl.ds(s * ZROWS, ZROWS)])
    pltpu.sync_copy(zeros.at[pl.ds(0, 5), :], degS.at[pl.ds(s * 5, 5)])
    pltpu.sync_copy(zeros.at[pl.ds(0, 5), :], c0S.at[pl.ds(s * 5, 5)])
    pltpu.sync_copy(zeros.at[pl.ds(0, 5), :], sS.at[pl.ds(s * 5, 5)])
    pltpu.sync_copy(zeros.at[pl.ds(0, NR), :], hA)
    pltpu.sync_copy(zeros.at[pl.ds(0, NR), :], hB)
    for k in range(NR // 16):
        idxb[pl.ds(k * 16, 16)] = i16 + k * 16

    def hblk(bi, _):
        base = s * EPT + bi * EBLK
        pltpu.sync_copy(adj.at[pl.ds((c * 2) * E + base, EBLK)], srcb)
        pltpu.sync_copy(adj.at[pl.ds((c * 2 + 1) * E + base, EBLK)], dstb)

        def chunk(i, _):
            d = dstb[pl.ds(i * 16, 16)]
            plsc.addupdate_scatter(hA, [d >> 7, d & 127], ones, mask=tmask)
            sr = srcb[pl.ds(i * 16, 16)]
            plsc.addupdate_scatter(hB, [sr >> 7, sr & 127], ones, mask=d == 0)
            return 0

        lax.fori_loop(0, EBLK // 16, chunk, 0)
        return 0

    lax.fori_loop(0, NBLK, hblk, 0)
    plsc.subcore_barrier()
    pltpu.sync_copy(hA, degS.at[idxb], add=True)
    pltpu.sync_copy(hB, c0S.at[idxb], add=True)
    plsc.subcore_barrier()

    pltpu.sync_copy(degS.at[pl.ds(s * 5, 5)], rowbuf.at[pl.ds(0, 5)])
    pltpu.sync_copy(c0S.at[pl.ds(s * 5, 5)], rowbuf.at[pl.ds(8, 5)])
    nbase = s * SLICE
    for k in range(SLICE // 16):
        r, l = k // 8, (k % 8) * 16
        deg16 = rowbuf[r, pl.ds(l, 16)]
        c016 = rowbuf[8 + r, pl.ds(l, 16)]
        nid = nbase + k * 16 + i16
        degf = deg16 + 1.0
        yi = plsc.bitcast(degf, jnp.int32)
        yi = 0x5F3759DF - (yi >> 1)
        rs = plsc.bitcast(yi, jnp.float32)
        for _ in range(3):
            rs = rs * (1.5 - 0.5 * degf * rs * rs)
        dinv16 = jnp.where(nid < N, rs, 0.0)
        c0f = c016 + jnp.where(nid == 0, 1.0, 0.0)
        ap16 = dinv16 * c0f
        rowbuf[r, pl.ds(l, 16)] = dinv16
        rowbuf[8 + r, pl.ds(l, 16)] = ap16
    pltpu.sync_copy(rowbuf.at[pl.ds(0, 5)], dinv_out.at[c, pl.ds(s * 5, 5), :])
    pltpu.sync_copy(rowbuf.at[pl.ds(8, 5)], ap_out.at[c, pl.ds(s * 5, 5), :])
    plsc.subcore_barrier()
    pltpu.sync_copy(dinv_out.at[c], hB)
    pltpu.sync_copy(ap_out.at[c], hC)
    pltpu.sync_copy(zeros.at[pl.ds(0, NR), :], hA)

    def make_pchunk(limit):
        def pchunk(w, _):
            off = w * 16
            pk = wl[pl.ds(off, 16)]
            lm = (off + i16) < limit
            src16 = pk & 16383
            dst16 = (pk >> 14) & 16383
            _process16(c, src16, jnp.where(lm, dst16, NP), lm,
                       xflat, hA, hB, rowbuf, dsvb, fence, gxs, sem)
            return 0

        return pchunk

    def blk(bi, wpos):
        base = s * EPT + bi * EBLK
        pltpu.sync_copy(adj.at[pl.ds((c * 2) * E + base, EBLK)], srcb)
        pltpu.sync_copy(adj.at[pl.ds((c * 2 + 1) * E + base, EBLK)], dstb)

        def group(g, wp):
            dsts = [dstb[pl.ds((g * GU + u) * 16, 16)] for u in range(GU)]
            avs = [plsc.load_gather(hC, [d >> 7, d & 127]) for d in dsts]
            m = avs[0]
            for u in range(1, GU):
                m = jnp.maximum(m, avs[u])
            hit = jnp.max(m)

            def on_hit(wp):
                for u in range(GU):
                    av = avs[u]
                    dst16 = dsts[u]
                    mk = av > 0.0
                    cnt = jnp.max(plsc.all_reduce_population_count(mk))

                    def flush(wpf):
                        lax.fori_loop(0, (wpf + 15) // 16, make_pchunk(wpf), 0)
                        return jnp.int32(0)

                    wp = lax.cond(wp + cnt > WL, flush, lambda w: w, wp)

                    def enqueue():
                        src16 = srcb[pl.ds((g * GU + u) * 16, 16)]
                        packed = src16 | (dst16 << 14)
                        plsc.store_compressed(
                            wl.at[pl.ds(wp, 16)], packed, mask=mk
                        )

                    pl.when(cnt > 0)(enqueue)
                    wp = wp + cnt
                return wp

            return lax.cond(hit > 0.0, on_hit, lambda wp: wp, wp)

        return lax.fori_loop(0, EBLK // 16 // GU, group, wpos)

    wpos = lax.fori_loop(0, NBLK, blk, jnp.int32(0))

    lax.fori_loop(0, (wpos + 15) // 16, make_pchunk(wpos), 0)
    plsc.subcore_barrier()
    pltpu.sync_copy(hA, sS.at[idxb], add=True)
    plsc.subcore_barrier()
    pltpu.sync_copy(sS.at[pl.ds(s * 5, 5)], svec_out.at[c, pl.ds(s * 5, 5), :])
    pltpu.sync_copy(
        gxs.at[pl.ds(s * SLICE, SLICE)], gxr_out.at[c, pl.ds(s * SLICE, SLICE), :]
    )


def _sc_mega(adj, xflat, zeros):
    f32 = jnp.float32
    kern = pl.kernel(
        _mega_body,
        out_type=[
            jax.ShapeDtypeStruct((B, NP, D), f32),
            jax.ShapeDtypeStruct((B, NR, 128), f32),
            jax.ShapeDtypeStruct((B, NR, 128), f32),
            jax.ShapeDtypeStruct((B, NR, 128), f32),
            jax.ShapeDtypeStruct((16, D), f32),
        ],
        mesh=_sc_mesh(),
        compiler_params=_sc_params(),
        scratch_types=[
            pltpu.VMEM((EBLK,), jnp.int32),
            pltpu.VMEM((EBLK,), jnp.int32),
            pltpu.VMEM((NR, 128), f32),
            pltpu.VMEM((NR, 128), f32),
            pltpu.VMEM((NR, 128), f32),
            pltpu.VMEM((16, D), f32),
            pltpu.VMEM((16,), f32),
            pltpu.VMEM((WL + 16,), jnp.int32),
            pltpu.VMEM((NR,), jnp.int32),
            pltpu.VMEM_SHARED((NR, 128), f32),
            pltpu.VMEM_SHARED((NR, 128), f32),
            pltpu.VMEM_SHARED((NR, 128), f32),
            pltpu.VMEM_SHARED((NP2, D), f32),
            pltpu.SemaphoreType.DMA,
        ],
    )
    return kern(adj, xflat, zeros)


def _final_body(gxr_ref, x_ref, dinv_ref, ap_ref, sv_ref, w1_ref, b1_ref,
                w2_ref, b2_ref, out_ref, u_s, sc_s):
    r = pl.program_id(1)
    nb = pl.num_programs(1)
    dinv = dinv_ref[0]
    ap = ap_ref[0]
    gx = gxr_ref[0]
    gs = sv_ref[0][0][:, None]
    xb = x_ref[0]
    dv = dinv[0][:, None]
    z = dv * gx + (dv * dv) * xb
    sv = dv * gs + dv * dv
    agg = jnp.dot(z, w1_ref[...], preferred_element_type=jnp.float32)
    agg = agg + sv * b1_ref[...]
    rl = jnp.maximum(agg, 0.0)
    upart = jnp.dot(ap, rl, preferred_element_type=jnp.float32)

    @pl.when(r == 0)
    def _():
        u_s[...] = jnp.zeros_like(u_s)
        sc_s[0] = 0.0
        sc_s[1] = dinv[0, 0]

    u_s[...] += upart
    sc_s[0] += jnp.sum(ap)

    @pl.when(r == nb - 1)
    def _():
        d0 = sc_s[1]
        o = jnp.dot(u_s[...], w2_ref[...], preferred_element_type=jnp.float32)
        out_ref[0] = d0 * o + (d0 * sc_s[0]) * b2_ref[...]


def _tc_final(gxr, xpad, dinv, ap, svec, W1, b1r, W2p, b2p):
    f32 = jnp.float32
    BN = 1024
    grid = (B, NP // BN)
    return pl.pallas_call(
        _final_body,
        grid=grid,
        in_specs=[
            pl.BlockSpec((1, BN, D), lambda b, r: (b, r, 0)),
            pl.BlockSpec((1, BN, D), lambda b, r: (b, r, 0)),
            pl.BlockSpec((1, 1, BN), lambda b, r: (b, 0, r)),
            pl.BlockSpec((1, 1, BN), lambda b, r: (b, 0, r)),
            pl.BlockSpec((1, 1, BN), lambda b, r: (b, 0, r)),
            pl.BlockSpec((D, H), lambda b, r: (0, 0)),
            pl.BlockSpec((1, H), lambda b, r: (0, 0)),
            pl.BlockSpec((H, 128), lambda b, r: (0, 0)),
            pl.BlockSpec((1, 128), lambda b, r: (0, 0)),
        ],
        out_specs=pl.BlockSpec((1, 1, 128), lambda b, r: (b, 0, 0)),
        out_shape=jax.ShapeDtypeStruct((B, 1, 128), f32),
        scratch_shapes=[
            pltpu.VMEM((1, H), f32),
            pltpu.SMEM((2,), f32),
        ],
    )(gxr, xpad, dinv, ap, svec, W1, b1r, W2p, b2p)


def kernel(adj, sen_adj, entity_adj, total_graph, sen_graph, entity_graph, x,
           lable, NQ, is_training, W1, b1, W2, b2):
    adj = adj.astype(jnp.int32).reshape(B * 2 * E)
    xpad = jnp.pad(x, ((0, 0), (0, NP - N), (0, 0)))
    xflat = xpad.reshape(B * NP, D)
    zeros = jnp.zeros((ZROWS, 128), jnp.float32)

    gxr, dinv3, ap3, svec3, _ = _sc_mega(adj, xflat, zeros)
    dinv = dinv3.reshape(B, 1, NP)
    ap = ap3.reshape(B, 1, NP)
    svec = svec3.reshape(B, 1, NP)

    b1r = b1.reshape(1, H)
    W2p = jnp.pad(W2, ((0, 0), (0, 128 - OUT)))
    b2p = jnp.pad(b2, (0, 128 - OUT)).reshape(1, 128)
    outp = _tc_final(gxr, xpad, dinv, ap, svec, W1, b1r, W2p, b2p)
    return outp[:, 0, :OUT]
